# baseline (speedup 1.0000x reference)
import functools

import jax
import jax.numpy as jnp
from jax import lax
from jax.experimental import pallas as pl
from jax.experimental.pallas import tpu as pltpu
from jax.experimental.pallas import tpu_sc as plsc

jax.config.update("jax_enable_x64", True)

L_SEQ = 32768
NW = 2 * L_SEQ
NS = 16
CHUNKW = NW // NS
LANES = 16
BYTE_BASE = 10
NEG = -(2 ** 30)


def _sc_body(tok_hbm, aux_hbm, best_hbm, out_hbm,
             chunk_v, aux_v, best_v, stage_v, win_v, out_v):
    sid = lax.axis_index("s")
    base_w = sid * CHUNKW
    pltpu.sync_copy(tok_hbm.at[pl.ds(base_w, CHUNKW)], chunk_v)
    pltpu.sync_copy(aux_hbm, aux_v)
    posaux = aux_v[pl.ds(0, LANES)]
    m = aux_v[pl.ds(LANES, LANES)]

    @plsc.parallel_loop(jnp.int32(0), jnp.int32(CHUNKW), jnp.int32(LANES),
                        unroll=4, carry=jnp.full((LANES,), -1, jnp.int32))
    def best(i, acc):
        v = chunk_v[pl.ds(i, LANES)]
        gidx = posaux + ((base_w + i) >> 1)
        return jnp.maximum(acc, jnp.where(v == m, gidx, jnp.int32(-1)))

    best_v[...] = best
    pltpu.sync_copy(best_v, best_hbm.at[pl.ds(sid * LANES, LANES)])
    plsc.subcore_barrier()

    @pl.when(sid == 0)
    def _():
        pltpu.sync_copy(best_hbm, stage_v)
        red = stage_v[pl.ds(0, LANES)]
        for i in range(1, NS):
            red = jnp.maximum(red, stage_v[pl.ds(i * LANES, LANES)])
        pos = jnp.max(red)
        found = pos >= 0
        pos0 = jnp.maximum(pos, 0)
        wstart = pl.multiple_of(
            jnp.minimum((2 * (pos0 + 1)) & ~7, NW - LANES), 8)
        pltpu.sync_copy(tok_hbm.at[pl.ds(wstart, LANES)], win_v)
        lane = lax.broadcasted_iota(jnp.int32, (LANES,), 0)
        k = jnp.minimum(lane, 3)
        local = 2 * jnp.clip(pos0 + 1 + k, 0, L_SEQ - 1) - wstart
        toks = plsc.load_gather(win_v, [local])
        byte_vals = jnp.clip(toks - jnp.int32(BYTE_BASE), 0, 255)
        zero = jnp.int32(0)
        mult = (jnp.where(lane == 0, jnp.int32(1), zero)
                + jnp.where(lane == 1, jnp.int32(256), zero)
                + jnp.where(lane == 2, jnp.int32(65536), zero)
                + jnp.where(lane == 3, jnp.int32(16777216), zero))
        value = jnp.sum(byte_vals * mult, dtype=jnp.int32)
        found_i32 = jnp.where(found, jnp.int32(1), zero)
        out_v[...] = (jnp.where(lane == 0, value, zero)
                      + jnp.where(lane == 1, found_i32, zero))
        pltpu.sync_copy(out_v, out_hbm)


@functools.partial(
    pl.kernel,
    out_type=(jax.ShapeDtypeStruct((NS * LANES,), jnp.int32),
              jax.ShapeDtypeStruct((LANES,), jnp.int32)),
    mesh=plsc.VectorSubcoreMesh(core_axis_name="c", subcore_axis_name="s",
                                num_cores=1, num_subcores=NS),
    scratch_types=[
        pltpu.VMEM((CHUNKW,), jnp.int32),
        pltpu.VMEM((2 * LANES,), jnp.int32),
        pltpu.VMEM((LANES,), jnp.int32),
        pltpu.VMEM((NS * LANES,), jnp.int32),
        pltpu.VMEM((LANES,), jnp.int32),
        pltpu.VMEM((LANES,), jnp.int32),
    ],
    compiler_params=pltpu.CompilerParams(needs_layout_passes=False),
)
def _reader_kernel(tok_hbm, aux_hbm, best_hbm, out_hbm, *scratch):
    _sc_body(tok_hbm, aux_hbm, best_hbm, out_hbm, *scratch)


def kernel(context_tokens, marker):
    words = lax.bitcast_convert_type(
        context_tokens[0], jnp.int32).reshape(NW)
    lanes = jnp.arange(LANES, dtype=jnp.int32)
    posaux = jnp.where(lanes % 2 == 0, lanes // 2, jnp.int32(NEG))
    aux = jnp.concatenate(
        [posaux, jnp.full((LANES,), marker, dtype=jnp.int32)])
    _, out = _reader_kernel(words, aux)
    val = out[0].astype(jnp.int64) & jnp.int64(4294967295)
    return jnp.where(out[1] > 0, val, jnp.int64(0))


# speedup vs baseline: 2.0209x; 2.0209x over previous
import functools

import jax
import jax.numpy as jnp
from jax import lax
from jax.experimental import pallas as pl
from jax.experimental.pallas import tpu as pltpu
from jax.experimental.pallas import tpu_sc as plsc

jax.config.update("jax_enable_x64", True)

L_SEQ = 32768
NS = 16
CHUNK = L_SEQ // NS
LANES = 16
BYTE_BASE = 10


def _sc_body(tok_hbm, marker_hbm, best_hbm, out_hbm,
             chunk_v, marker_v, best_v, stage_v, win_v, out_v):
    sid = lax.axis_index("s")
    base = sid * CHUNK
    pltpu.sync_copy(tok_hbm.at[pl.ds(base, CHUNK)], chunk_v)
    pltpu.sync_copy(marker_hbm, marker_v)
    m = marker_v[...]
    lane = lax.broadcasted_iota(jnp.int32, (LANES,), 0)

    @plsc.parallel_loop(jnp.int32(0), jnp.int32(CHUNK), jnp.int32(LANES),
                        unroll=8, carry=jnp.full((LANES,), -1, jnp.int32))
    def best(i, acc):
        v = chunk_v[pl.ds(i, LANES)]
        gidx = lane + (base + i)
        return jnp.maximum(acc, jnp.where(v == m, gidx, jnp.int32(-1)))

    best_v[...] = best
    pltpu.sync_copy(best_v, best_hbm.at[pl.ds(sid * LANES, LANES)])
    plsc.subcore_barrier()

    @pl.when(sid == 0)
    def _():
        pltpu.sync_copy(best_hbm, stage_v)
        red = stage_v[pl.ds(0, LANES)]
        for i in range(1, NS):
            red = jnp.maximum(red, stage_v[pl.ds(i * LANES, LANES)])
        pos = jnp.max(red)
        found = pos >= 0
        pos0 = jnp.maximum(pos, 0)
        wstart = pl.multiple_of(
            jnp.minimum((pos0 + 1) & ~7, L_SEQ - LANES), 8)
        pltpu.sync_copy(tok_hbm.at[pl.ds(wstart, LANES)], win_v)
        k = jnp.minimum(lane, 3)
        local = jnp.clip(pos0 + 1 + k, 0, L_SEQ - 1) - wstart
        toks = plsc.load_gather(win_v, [local])
        byte_vals = jnp.clip(toks - jnp.int32(BYTE_BASE), 0, 255)
        zero = jnp.int32(0)
        mult = (jnp.where(lane == 0, jnp.int32(1), zero)
                + jnp.where(lane == 1, jnp.int32(256), zero)
                + jnp.where(lane == 2, jnp.int32(65536), zero)
                + jnp.where(lane == 3, jnp.int32(16777216), zero))
        value = jnp.sum(byte_vals * mult, dtype=jnp.int32)
        value = value * jnp.where(found, jnp.int32(1), zero)
        out_v[...] = jnp.where(lane == 0, value, zero)
        pltpu.sync_copy(out_v, out_hbm)


@functools.partial(
    pl.kernel,
    out_type=(jax.ShapeDtypeStruct((NS * LANES,), jnp.int32),
              jax.ShapeDtypeStruct((LANES,), jnp.int32)),
    mesh=plsc.VectorSubcoreMesh(core_axis_name="c", subcore_axis_name="s",
                                num_cores=1, num_subcores=NS),
    scratch_types=[
        pltpu.VMEM((CHUNK,), jnp.int32),
        pltpu.VMEM((LANES,), jnp.int32),
        pltpu.VMEM((LANES,), jnp.int32),
        pltpu.VMEM((NS * LANES,), jnp.int32),
        pltpu.VMEM((LANES,), jnp.int32),
        pltpu.VMEM((LANES,), jnp.int32),
    ],
    compiler_params=pltpu.CompilerParams(needs_layout_passes=False),
)
def _reader_kernel(tok_hbm, marker_hbm, best_hbm, out_hbm, *scratch):
    _sc_body(tok_hbm, marker_hbm, best_hbm, out_hbm, *scratch)


def kernel(context_tokens, marker):
    tok32 = context_tokens[0].astype(jnp.int32)
    marker_arr = jnp.full((LANES,), marker, dtype=jnp.int32)
    _, out = _reader_kernel(tok32, marker_arr)
    return out[0].astype(jnp.int64) & jnp.int64(4294967295)


# speedup vs baseline: 5.6966x; 2.8188x over previous
import functools

import jax
import jax.numpy as jnp
from jax import lax
from jax.experimental import pallas as pl
from jax.experimental.pallas import tpu as pltpu

jax.config.update("jax_enable_x64", True)

L_SEQ = 32768
ROWS = 256
COLS = 128
BYTE_BASE = 10


def _tc_body(marker_ref, tok_ref, out_ref):
    x = tok_ref[...]
    m = marker_ref[0, 0]
    row = lax.broadcasted_iota(jnp.int32, (ROWS, COLS), 0)
    col = lax.broadcasted_iota(jnp.int32, (ROWS, COLS), 1)
    idx = row * COLS + col
    scores = jnp.where(x == m, idx, jnp.int32(-1))
    pos = jnp.max(scores)
    found = pos >= 0
    pos0 = jnp.maximum(pos, 0)
    value = jnp.int32(0)
    mults = (1, 256, 65536, 16777216)
    for k in range(4):
        t = jnp.clip(pos0 + jnp.int32(1 + k), 0, L_SEQ - 1)
        tok = jnp.max(jnp.where(idx == t, x, jnp.int32(0)))
        byte = jnp.clip(tok - jnp.int32(BYTE_BASE), 0, 255)
        value = value + byte * jnp.int32(mults[k])
    value = value * jnp.where(found, jnp.int32(1), jnp.int32(0))
    out_ref[...] = jnp.full((8, COLS), value, jnp.int32)


@functools.partial(jax.jit, static_argnames=())
def kernel(context_tokens, marker):
    tok32 = context_tokens[0].astype(jnp.int32).reshape(ROWS, COLS)
    marker_arr = jnp.asarray(marker, jnp.int32).reshape(1, 1)
    out = pl.pallas_call(
        _tc_body,
        out_shape=jax.ShapeDtypeStruct((8, COLS), jnp.int32),
        in_specs=[
            pl.BlockSpec(memory_space=pltpu.SMEM),
            pl.BlockSpec(memory_space=pltpu.ANY if False else pltpu.VMEM),
        ],
        out_specs=pl.BlockSpec(memory_space=pltpu.VMEM),
    )(marker_arr, tok32)
    return out[0, 0].astype(jnp.int64) & jnp.int64(4294967295)


# speedup vs baseline: 6.8319x; 1.1993x over previous
import jax
import jax.numpy as jnp
from jax import lax
from jax.experimental import pallas as pl
from jax.experimental.pallas import tpu as pltpu

jax.config.update("jax_enable_x64", True)

L_SEQ = 32768
ROWS = 256
COLS = 128
BYTE_BASE = 10


def _tc_body(marker_ref, tok_ref, out_ref):
    x = tok_ref[...]
    m = marker_ref[0, 0]
    row = lax.broadcasted_iota(jnp.int32, (ROWS, COLS), 0)
    col = lax.broadcasted_iota(jnp.int32, (ROWS, COLS), 1)
    idx = row * COLS + col
    scores = jnp.where(x == m, idx, jnp.int32(-1))
    pos = jnp.max(scores)
    found = pos >= 0
    pos0 = jnp.maximum(pos, 0)
    r0 = jnp.minimum((pos0 + 1) // COLS, ROWS - 2)
    win = tok_ref[pl.ds(r0, 2), :]
    wrow = lax.broadcasted_iota(jnp.int32, (2, COLS), 0)
    wcol = lax.broadcasted_iota(jnp.int32, (2, COLS), 1)
    widx = (r0 + wrow) * COLS + wcol
    value = jnp.int32(0)
    mults = (1, 256, 65536, 16777216)
    for k in range(4):
        t = jnp.clip(pos0 + jnp.int32(1 + k), 0, L_SEQ - 1)
        tok = jnp.max(jnp.where(widx == t, win, jnp.int32(0)))
        byte = jnp.clip(tok - jnp.int32(BYTE_BASE), 0, 255)
        value = value + byte * jnp.int32(mults[k])
    value = value * jnp.where(found, jnp.int32(1), jnp.int32(0))
    out_ref[0, 0] = value


def kernel(context_tokens, marker):
    tok32 = context_tokens[0].astype(jnp.int32).reshape(ROWS, COLS)
    marker_arr = jnp.asarray(marker, jnp.int32).reshape(1, 1)
    out = pl.pallas_call(
        _tc_body,
        out_shape=jax.ShapeDtypeStruct((1, 1), jnp.int32),
        in_specs=[
            pl.BlockSpec(memory_space=pltpu.SMEM),
            pl.BlockSpec(memory_space=pltpu.VMEM),
        ],
        out_specs=pl.BlockSpec(memory_space=pltpu.SMEM),
    )(marker_arr, tok32)
    return out[0, 0].astype(jnp.int64) & jnp.int64(4294967295)
